# Initial kernel scaffold; baseline (speedup 1.0000x reference)
import math

import jax
import jax.numpy as jnp
from jax import lax
from jax.experimental import pallas as pl
from jax.experimental.pallas import tpu as pltpu
from jax.experimental.pallas import tpu_sc as plsc

N = 10000
D = 128
DE = 16
E = 320000
EPS = 1e-5

NP = 10240
EP = 327680
NC = 2
NS = 16
NW = NC * NS
EPT = EP // NW
CH = 128
NCH = EPT // CH
RPT = NP // NS
ROWS2D = EP // CH

BN_SCALE = 1.0 / math.sqrt(1.0 + EPS)

_mesh = plsc.VectorSubcoreMesh(core_axis_name="c", subcore_axis_name="s")


def _deg_body(row2d, ones_hbm, zer_hbm, out_hbm, idx_v, ones_v, acc_sh):
    c = lax.axis_index("c")
    s = lax.axis_index("s")
    wid = c * NS + s
    pltpu.sync_copy(zer_hbm, acc_sh.at[pl.ds(s * RPT, RPT)])
    pltpu.sync_copy(ones_hbm, ones_v)
    pltpu.sync_copy(row2d.at[pl.ds(wid * NCH, NCH)], idx_v)
    plsc.subcore_barrier()

    def body(i, carry):
        pltpu.sync_copy(ones_v, acc_sh.at[idx_v.at[i]], add=True)
        return carry

    lax.fori_loop(0, NCH, body, 0)
    plsc.subcore_barrier()
    pltpu.sync_copy(acc_sh.at[pl.ds(s * RPT, RPT)],
                    out_hbm.at[pl.ds(c * NP + s * RPT, RPT)])


_sc_deg = pl.kernel(
    _deg_body,
    out_type=jax.ShapeDtypeStruct((2 * NP, 16), jnp.float32),
    mesh=_mesh,
    scratch_types=[
        pltpu.VMEM((NCH, CH), jnp.int32),
        pltpu.VMEM((CH, 16), jnp.float32),
        pltpu.VMEM_SHARED((NP, 16), jnp.float32),
    ],
)


def _disrow_body(row2d, dis_hbm, out2d, idx_v, dis_v, out_v):
    c = lax.axis_index("c")
    s = lax.axis_index("s")
    wid = c * NS + s
    pltpu.sync_copy(dis_hbm, dis_v)
    pltpu.sync_copy(row2d.at[pl.ds(wid * NCH, NCH)], idx_v)

    def body(i, carry):
        for j in range(CH // 16):
            sl = pl.ds(j * 16, 16)
            out_v[i, sl] = plsc.load_gather(dis_v, [idx_v[i, sl]])
        return carry

    lax.fori_loop(0, NCH, body, 0)
    pltpu.sync_copy(out_v, out2d.at[pl.ds(wid * NCH, NCH)])


_sc_disrow = pl.kernel(
    _disrow_body,
    out_type=jax.ShapeDtypeStruct((ROWS2D, CH), jnp.float32),
    mesh=_mesh,
    scratch_types=[
        pltpu.VMEM((NCH, CH), jnp.int32),
        pltpu.VMEM((NP,), jnp.float32),
        pltpu.VMEM((NCH, CH), jnp.float32),
    ],
)


def _gcn_body(xs_hbm, emb_hbm, row2d, col2d, zer_hbm, out_hbm,
              idxr, idxc, xg, ev, acc_sh, sem):
    c = lax.axis_index("c")
    s = lax.axis_index("s")
    wid = c * NS + s
    pltpu.sync_copy(zer_hbm, acc_sh.at[pl.ds(s * RPT, RPT)])
    pltpu.sync_copy(row2d.at[pl.ds(wid * NCH, NCH)], idxr)
    pltpu.sync_copy(col2d.at[pl.ds(wid * NCH, NCH)], idxc)
    plsc.subcore_barrier()

    def chunk(i, carry):
        pltpu.async_copy(xs_hbm.at[idxr.at[i]], xg, sem).wait()
        pltpu.sync_copy(emb_hbm.at[pl.ds(wid * EPT + i * CH, CH)], ev)

        def rowbody(r, cc):
            for j in range(D // 16):
                sl = pl.ds(j * 16, 16)
                xg[r, sl] = jnp.maximum(xg[r, sl] + ev[r, sl], 0.0)
            return cc

        lax.fori_loop(0, CH, rowbody, 0)
        pltpu.sync_copy(xg, acc_sh.at[idxc.at[i]], add=True)
        return carry

    lax.fori_loop(0, NCH, chunk, 0)
    plsc.subcore_barrier()
    pltpu.sync_copy(acc_sh.at[pl.ds(s * RPT, RPT)],
                    out_hbm.at[pl.ds(c * NP + s * RPT, RPT)])


_sc_gcn = pl.kernel(
    _gcn_body,
    out_type=jax.ShapeDtypeStruct((2 * NP, D), jnp.float32),
    mesh=_mesh,
    scratch_types=[
        pltpu.VMEM((NCH, CH), jnp.int32),
        pltpu.VMEM((NCH, CH), jnp.int32),
        pltpu.VMEM((CH, D), jnp.float32),
        pltpu.VMEM((CH, D), jnp.float32),
        pltpu.VMEM_SHARED((NP, D), jnp.float32),
        pltpu.SemaphoreType.DMA,
    ],
)


BNR = 1024
BER = 4096


def _enc_body(x_ref, wT_ref, b_ref, o_ref):
    o_ref[...] = jnp.maximum(
        jnp.dot(x_ref[...], wT_ref[...], preferred_element_type=jnp.float32)
        + b_ref[...], 0.0)


_tc_enc = pl.pallas_call(
    _enc_body,
    out_shape=jax.ShapeDtypeStruct((NP, D), jnp.float32),
    grid=(NP // BNR,),
    in_specs=[
        pl.BlockSpec((BNR, D), lambda i: (i, 0)),
        pl.BlockSpec((D, D), lambda i: (0, 0)),
        pl.BlockSpec((1, D), lambda i: (0, 0)),
    ],
    out_specs=pl.BlockSpec((BNR, D), lambda i: (i, 0)),
)


def _nodemm_body(h_ref, wT_ref, b_ref, dis_ref, y_ref, ys_ref):
    y = (jnp.dot(h_ref[...], wT_ref[...], preferred_element_type=jnp.float32)
         + b_ref[...])
    y_ref[...] = y
    ys_ref[...] = y * dis_ref[...]


_tc_nodemm = pl.pallas_call(
    _nodemm_body,
    out_shape=(jax.ShapeDtypeStruct((NP, D), jnp.float32),
               jax.ShapeDtypeStruct((NP, D), jnp.float32)),
    grid=(NP // BNR,),
    in_specs=[
        pl.BlockSpec((BNR, D), lambda i: (i, 0)),
        pl.BlockSpec((D, D), lambda i: (0, 0)),
        pl.BlockSpec((1, D), lambda i: (0, 0)),
        pl.BlockSpec((BNR, 1), lambda i: (i, 0)),
    ],
    out_specs=(pl.BlockSpec((BNR, D), lambda i: (i, 0)),
               pl.BlockSpec((BNR, D), lambda i: (i, 0))),
)


def _edgemm_body(ea_ref, eWT_ref, eb_ref, dr_ref, o_ref):
    m = (jnp.dot(ea_ref[...], eWT_ref[...], preferred_element_type=jnp.float32)
         + eb_ref[...])
    o_ref[...] = m * dr_ref[...]


_tc_edgemm = pl.pallas_call(
    _edgemm_body,
    out_shape=jax.ShapeDtypeStruct((EP, D), jnp.float32),
    grid=(EP // BER,),
    in_specs=[
        pl.BlockSpec((BER, DE), lambda i: (i, 0)),
        pl.BlockSpec((DE, D), lambda i: (0, 0)),
        pl.BlockSpec((1, D), lambda i: (0, 0)),
        pl.BlockSpec((BER, 1), lambda i: (i, 0)),
    ],
    out_specs=pl.BlockSpec((BER, D), lambda i: (i, 0)),
)


def _prep_body(d0_ref, d1_ref, dis_ref, inv_ref):
    deg = d0_ref[...][:, :1] + d1_ref[...][:, :1] + 1.0
    dis_ref[...] = lax.rsqrt(deg)
    inv_ref[...] = 1.0 / deg


_tc_prep = pl.pallas_call(
    _prep_body,
    out_shape=(jax.ShapeDtypeStruct((NP, 1), jnp.float32),
               jax.ShapeDtypeStruct((NP, 1), jnp.float32)),
    grid=(NP // BNR,),
    in_specs=[
        pl.BlockSpec((BNR, 16), lambda i: (i, 0)),
        pl.BlockSpec((BNR, 16), lambda i: (i, 0)),
    ],
    out_specs=(pl.BlockSpec((BNR, 1), lambda i: (i, 0)),
               pl.BlockSpec((BNR, 1), lambda i: (i, 0))),
)


def _make_combine(do_relu):
    def _combine_body(a0_ref, a1_ref, y_ref, dis_ref, inv_ref, g_ref, bb_ref,
                      o_ref):
        y = y_ref[...]
        agg = dis_ref[...] * (a0_ref[...] + a1_ref[...]) \
            + jnp.maximum(y, 0.0) * inv_ref[...]
        r = agg * (g_ref[...] * BN_SCALE) + bb_ref[...]
        if do_relu:
            r = jnp.maximum(r, 0.0)
        o_ref[...] = r

    return pl.pallas_call(
        _combine_body,
        out_shape=jax.ShapeDtypeStruct((NP, D), jnp.float32),
        grid=(NP // BNR,),
        in_specs=[
            pl.BlockSpec((BNR, D), lambda i: (i, 0)),
            pl.BlockSpec((BNR, D), lambda i: (i, 0)),
            pl.BlockSpec((BNR, D), lambda i: (i, 0)),
            pl.BlockSpec((BNR, 1), lambda i: (i, 0)),
            pl.BlockSpec((BNR, 1), lambda i: (i, 0)),
            pl.BlockSpec((1, D), lambda i: (0, 0)),
            pl.BlockSpec((1, D), lambda i: (0, 0)),
        ],
        out_specs=pl.BlockSpec((BNR, D), lambda i: (i, 0)),
    )


_tc_combine_relu = _make_combine(True)
_tc_combine_last = _make_combine(False)


def kernel(x, edge_index, edge_attr, node_W, node_b,
           conv0_W, conv0_b, conv0_eW, conv0_eb, bn0_g, bn0_b,
           conv1_W, conv1_b, conv1_eW, conv1_eb, bn1_g, bn1_b,
           conv2_W, conv2_b, conv2_eW, conv2_eb, bn2_g, bn2_b):
    f32 = jnp.float32
    row = edge_index[0].astype(jnp.int32)
    col = edge_index[1].astype(jnp.int32)
    pad_idx = jnp.full((EP - E,), N, jnp.int32)
    row2d = jnp.concatenate([row, pad_idx]).reshape(ROWS2D, CH)
    col2d = jnp.concatenate([col, pad_idx]).reshape(ROWS2D, CH)
    eap = jnp.pad(edge_attr.astype(f32), ((0, EP - E), (0, 0)))
    xp = jnp.pad(x.astype(f32), ((0, NP - N), (0, 0)))

    ones16 = jnp.ones((CH, 16), f32)
    zer16 = jnp.zeros((RPT, 16), f32)
    zerD = jnp.zeros((RPT, D), f32)

    deg2 = _sc_deg(row2d, ones16, zer16)
    dis, inv = _tc_prep(deg2[:NP], deg2[NP:])
    dis_row = _sc_disrow(row2d, dis.reshape(NP)).reshape(EP, 1)

    h = _tc_enc(xp, node_W.T, node_b.reshape(1, D))

    convs = [
        (conv0_W, conv0_b, conv0_eW, conv0_eb, bn0_g, bn0_b, _tc_combine_relu),
        (conv1_W, conv1_b, conv1_eW, conv1_eb, bn1_g, bn1_b, _tc_combine_relu),
        (conv2_W, conv2_b, conv2_eW, conv2_eb, bn2_g, bn2_b, _tc_combine_last),
    ]
    for W, b, eW, eb, g, bb, combine in convs:
        y, ys = _tc_nodemm(h, W.T, b.reshape(1, D), dis)
        emb = _tc_edgemm(eap, eW.T, eb.reshape(1, D), dis_row)
        accs = _sc_gcn(ys, emb, row2d, col2d, zerD)
        h = combine(accs[:NP], accs[NP:], y, dis, inv,
                    g.reshape(1, D), bb.reshape(1, D))

    return h[:N]


# speedup vs baseline: 2.4775x; 2.4775x over previous
import math

import jax
import jax.numpy as jnp
from jax import lax
from jax.experimental import pallas as pl
from jax.experimental.pallas import tpu as pltpu
from jax.experimental.pallas import tpu_sc as plsc

N = 10000
D = 128
DE = 16
E = 320000
EPS = 1e-5

NP = 10240
EP = 327680
NC = 2
NS = 16
NW = NC * NS
EPT = EP // NW
CH = 128
NCH = EPT // CH
RPT = NP // NS
ROWS2D = EP // CH

BN_SCALE = 1.0 / math.sqrt(1.0 + EPS)

_mesh = plsc.VectorSubcoreMesh(core_axis_name="c", subcore_axis_name="s",
                               num_cores=NC, num_subcores=NS)


def _deg_body(row2d, ones_hbm, zer_hbm, out_hbm, idx_v, ones_v, acc_sh):
    c = lax.axis_index("c")
    s = lax.axis_index("s")
    wid = c * NS + s
    pltpu.sync_copy(zer_hbm, acc_sh.at[pl.ds(s * RPT, RPT)])
    pltpu.sync_copy(ones_hbm, ones_v)
    pltpu.sync_copy(row2d.at[pl.ds(wid * NCH, NCH)], idx_v)
    plsc.subcore_barrier()

    def body(i, carry):
        pltpu.sync_copy(ones_v, acc_sh.at[idx_v.at[i]], add=True)
        return carry

    lax.fori_loop(0, NCH, body, 0)
    plsc.subcore_barrier()
    pltpu.sync_copy(acc_sh.at[pl.ds(s * RPT, RPT)],
                    out_hbm.at[pl.ds(c * NP + s * RPT, RPT)])


_sc_deg = pl.kernel(
    _deg_body,
    out_type=jax.ShapeDtypeStruct((2 * NP, 16), jnp.float32),
    mesh=_mesh,
    scratch_types=[
        pltpu.VMEM((NCH, CH), jnp.int32),
        pltpu.VMEM((CH, 16), jnp.float32),
        pltpu.VMEM_SHARED((NP, 16), jnp.float32),
    ],
    compiler_params=pltpu.CompilerParams(use_tc_tiling_on_sc=False),
)


def _disrow_body(dis16_hbm, row2d, out16, idx_v, buf, sem):
    c = lax.axis_index("c")
    s = lax.axis_index("s")
    wid = c * NS + s
    pltpu.sync_copy(row2d.at[pl.ds(wid * NCH, NCH)], idx_v)

    def body(i, carry):
        pltpu.async_copy(dis16_hbm.at[idx_v.at[i]], buf, sem).wait()
        pltpu.sync_copy(buf, out16.at[pl.ds(wid * EPT + i * CH, CH)])
        return carry

    lax.fori_loop(0, NCH, body, 0)


_sc_disrow = pl.kernel(
    _disrow_body,
    out_type=jax.ShapeDtypeStruct((EP, 16), jnp.float32),
    mesh=_mesh,
    scratch_types=[
        pltpu.VMEM((NCH, CH), jnp.int32),
        pltpu.VMEM((CH, 16), jnp.float32),
        pltpu.SemaphoreType.DMA,
    ],
    compiler_params=pltpu.CompilerParams(use_tc_tiling_on_sc=False),
)


DH = D // 2


def _gcn_body(xs_hbm, emb_hbm, row2d, col2d, zer_hbm, out_hbm,
              idxr, idxc, xg, ev, acc_sh, sem):
    c = lax.axis_index("c")
    s = lax.axis_index("s")
    wid = c * NS + s

    def zbody(k, carry):
        pltpu.sync_copy(zer_hbm, acc_sh.at[pl.ds(s * RPT + k * CH, CH)])
        return carry

    lax.fori_loop(0, RPT // CH, zbody, 0)
    pltpu.sync_copy(row2d.at[pl.ds(wid * NCH, NCH)], idxr)
    pltpu.sync_copy(col2d.at[pl.ds(wid * NCH, NCH)], idxc)
    plsc.subcore_barrier()

    def chunk(i, carry):
        pltpu.async_copy(xs_hbm.at[idxr.at[i]], xg, sem).wait()
        pltpu.sync_copy(emb_hbm.at[pl.ds(wid * EPT + i * CH, CH)], ev)

        def rowbody(r, cc):
            for j in range(DH // 16):
                sl = pl.ds(j * 16, 16)
                xg[r, sl] = jnp.maximum(xg[r, sl] + ev[r, sl], 0.0)
            return cc

        lax.fori_loop(0, CH, rowbody, 0)
        pltpu.sync_copy(xg, acc_sh.at[idxc.at[i]], add=True)
        return carry

    lax.fori_loop(0, NCH, chunk, 0)
    plsc.subcore_barrier()
    pltpu.sync_copy(acc_sh.at[pl.ds(s * RPT, RPT)],
                    out_hbm.at[pl.ds(c * NP + s * RPT, RPT)])


_sc_gcn = pl.kernel(
    _gcn_body,
    out_type=jax.ShapeDtypeStruct((2 * NP, DH), jnp.float32),
    mesh=_mesh,
    scratch_types=[
        pltpu.VMEM((NCH, CH), jnp.int32),
        pltpu.VMEM((NCH, CH), jnp.int32),
        pltpu.VMEM((CH, DH), jnp.float32),
        pltpu.VMEM((CH, DH), jnp.float32),
        pltpu.VMEM_SHARED((NP, DH), jnp.float32),
        pltpu.SemaphoreType.DMA,
    ],
    compiler_params=pltpu.CompilerParams(use_tc_tiling_on_sc=False),
)


BNR = 1024
BER = 4096


def _enc_body(x_ref, wT_ref, b_ref, o_ref):
    o_ref[...] = jnp.maximum(
        jnp.dot(x_ref[...], wT_ref[...], preferred_element_type=jnp.float32)
        + b_ref[...], 0.0)


_tc_enc = pl.pallas_call(
    _enc_body,
    out_shape=jax.ShapeDtypeStruct((NP, D), jnp.float32),
    grid=(NP // BNR,),
    in_specs=[
        pl.BlockSpec((BNR, D), lambda i: (i, 0)),
        pl.BlockSpec((D, D), lambda i: (0, 0)),
        pl.BlockSpec((1, D), lambda i: (0, 0)),
    ],
    out_specs=pl.BlockSpec((BNR, D), lambda i: (i, 0)),
)


def _nodemm_body(h_ref, wT_ref, b_ref, dis_ref, y_ref, ys0_ref, ys1_ref):
    y = (jnp.dot(h_ref[...], wT_ref[...], preferred_element_type=jnp.float32)
         + b_ref[...])
    y_ref[...] = y
    ys = y * dis_ref[...]
    ys0_ref[...] = ys[:, :DH]
    ys1_ref[...] = ys[:, DH:]


_tc_nodemm = pl.pallas_call(
    _nodemm_body,
    out_shape=(jax.ShapeDtypeStruct((NP, D), jnp.float32),
               jax.ShapeDtypeStruct((NP, DH), jnp.float32),
               jax.ShapeDtypeStruct((NP, DH), jnp.float32)),
    grid=(NP // BNR,),
    in_specs=[
        pl.BlockSpec((BNR, D), lambda i: (i, 0)),
        pl.BlockSpec((D, D), lambda i: (0, 0)),
        pl.BlockSpec((1, D), lambda i: (0, 0)),
        pl.BlockSpec((BNR, 1), lambda i: (i, 0)),
    ],
    out_specs=(pl.BlockSpec((BNR, D), lambda i: (i, 0)),
               pl.BlockSpec((BNR, DH), lambda i: (i, 0)),
               pl.BlockSpec((BNR, DH), lambda i: (i, 0))),
)


def _edgemm_body(ea_ref, eWT_ref, eb_ref, dr_ref, o0_ref, o1_ref):
    m = (jnp.dot(ea_ref[...], eWT_ref[...], preferred_element_type=jnp.float32)
         + eb_ref[...]) * dr_ref[...]
    o0_ref[...] = m[:, :DH]
    o1_ref[...] = m[:, DH:]


_tc_edgemm = pl.pallas_call(
    _edgemm_body,
    out_shape=(jax.ShapeDtypeStruct((EP, DH), jnp.float32),
               jax.ShapeDtypeStruct((EP, DH), jnp.float32)),
    grid=(EP // BER,),
    in_specs=[
        pl.BlockSpec((BER, DE), lambda i: (i, 0)),
        pl.BlockSpec((DE, D), lambda i: (0, 0)),
        pl.BlockSpec((1, D), lambda i: (0, 0)),
        pl.BlockSpec((BER, 1), lambda i: (i, 0)),
    ],
    out_specs=(pl.BlockSpec((BER, DH), lambda i: (i, 0)),
               pl.BlockSpec((BER, DH), lambda i: (i, 0))),
)


def _prep_body(d0_ref, d1_ref, dis_ref, inv_ref, dis16_ref):
    deg = d0_ref[...] + d1_ref[...] + 1.0
    dis16 = lax.rsqrt(deg)
    dis16_ref[...] = dis16
    dis_ref[...] = dis16[:, :1]
    inv_ref[...] = 1.0 / deg[:, :1]


_tc_prep = pl.pallas_call(
    _prep_body,
    out_shape=(jax.ShapeDtypeStruct((NP, 1), jnp.float32),
               jax.ShapeDtypeStruct((NP, 1), jnp.float32),
               jax.ShapeDtypeStruct((NP, 16), jnp.float32)),
    grid=(NP // BNR,),
    in_specs=[
        pl.BlockSpec((BNR, 16), lambda i: (i, 0)),
        pl.BlockSpec((BNR, 16), lambda i: (i, 0)),
    ],
    out_specs=(pl.BlockSpec((BNR, 1), lambda i: (i, 0)),
               pl.BlockSpec((BNR, 1), lambda i: (i, 0)),
               pl.BlockSpec((BNR, 16), lambda i: (i, 0))),
)


def _make_combine(do_relu):
    def _combine_body(a0l_ref, a0h_ref, a1l_ref, a1h_ref, y_ref, dis_ref,
                      inv_ref, g_ref, bb_ref, o_ref):
        y = y_ref[...]
        acc = jnp.concatenate([a0l_ref[...] + a1l_ref[...],
                               a0h_ref[...] + a1h_ref[...]], axis=1)
        agg = dis_ref[...] * acc + jnp.maximum(y, 0.0) * inv_ref[...]
        r = agg * (g_ref[...] * BN_SCALE) + bb_ref[...]
        if do_relu:
            r = jnp.maximum(r, 0.0)
        o_ref[...] = r

    return pl.pallas_call(
        _combine_body,
        out_shape=jax.ShapeDtypeStruct((NP, D), jnp.float32),
        grid=(NP // BNR,),
        in_specs=[
            pl.BlockSpec((BNR, DH), lambda i: (i, 0)),
            pl.BlockSpec((BNR, DH), lambda i: (i, 0)),
            pl.BlockSpec((BNR, DH), lambda i: (i, 0)),
            pl.BlockSpec((BNR, DH), lambda i: (i, 0)),
            pl.BlockSpec((BNR, D), lambda i: (i, 0)),
            pl.BlockSpec((BNR, 1), lambda i: (i, 0)),
            pl.BlockSpec((BNR, 1), lambda i: (i, 0)),
            pl.BlockSpec((1, D), lambda i: (0, 0)),
            pl.BlockSpec((1, D), lambda i: (0, 0)),
        ],
        out_specs=pl.BlockSpec((BNR, D), lambda i: (i, 0)),
    )


_tc_combine_relu = _make_combine(True)
_tc_combine_last = _make_combine(False)


def kernel(x, edge_index, edge_attr, node_W, node_b,
           conv0_W, conv0_b, conv0_eW, conv0_eb, bn0_g, bn0_b,
           conv1_W, conv1_b, conv1_eW, conv1_eb, bn1_g, bn1_b,
           conv2_W, conv2_b, conv2_eW, conv2_eb, bn2_g, bn2_b):
    f32 = jnp.float32
    row = edge_index[0].astype(jnp.int32)
    col = edge_index[1].astype(jnp.int32)
    pad_idx = jnp.full((EP - E,), N, jnp.int32)
    row2d = jnp.concatenate([row, pad_idx]).reshape(ROWS2D, CH)
    col2d = jnp.concatenate([col, pad_idx]).reshape(ROWS2D, CH)
    eap = jnp.pad(edge_attr.astype(f32), ((0, EP - E), (0, 0)))
    xp = jnp.pad(x.astype(f32), ((0, NP - N), (0, 0)))

    ones16 = jnp.ones((CH, 16), f32)
    zer16 = jnp.zeros((RPT, 16), f32)
    zerD = jnp.zeros((CH, DH), f32)

    deg2 = _sc_deg(row2d, ones16, zer16)
    dis, inv, dis16 = _tc_prep(deg2[:NP], deg2[NP:])
    dis_row = _sc_disrow(dis16, row2d)[:, :1]

    h = _tc_enc(xp, node_W.T, node_b.reshape(1, D))

    convs = [
        (conv0_W, conv0_b, conv0_eW, conv0_eb, bn0_g, bn0_b, _tc_combine_relu),
        (conv1_W, conv1_b, conv1_eW, conv1_eb, bn1_g, bn1_b, _tc_combine_relu),
        (conv2_W, conv2_b, conv2_eW, conv2_eb, bn2_g, bn2_b, _tc_combine_last),
    ]
    for W, b, eW, eb, g, bb, combine in convs:
        y, ys0, ys1 = _tc_nodemm(h, W.T, b.reshape(1, D), dis)
        emb0, emb1 = _tc_edgemm(eap, eW.T, eb.reshape(1, D), dis_row)
        acc_lo = _sc_gcn(ys0, emb0, row2d, col2d, zerD)
        acc_hi = _sc_gcn(ys1, emb1, row2d, col2d, zerD)
        h = combine(acc_lo[:NP], acc_hi[:NP], acc_lo[NP:], acc_hi[NP:],
                    y, dis, inv, g.reshape(1, D), bb.reshape(1, D))

    return h[:N]


# speedup vs baseline: 2.9765x; 1.2014x over previous
import math

import jax
import jax.numpy as jnp
from jax import lax
from jax.experimental import pallas as pl
from jax.experimental.pallas import tpu as pltpu
from jax.experimental.pallas import tpu_sc as plsc

N = 10000
D = 128
DE = 16
E = 320000
EPS = 1e-5

NP = 10240
EP = 327680
NC = 2
NS = 16
NW = NC * NS
EPT = EP // NW
CH = 128
NCH = EPT // CH
RPT = NP // NS
ROWS2D = EP // CH

BN_SCALE = 1.0 / math.sqrt(1.0 + EPS)

_mesh = plsc.VectorSubcoreMesh(core_axis_name="c", subcore_axis_name="s",
                               num_cores=NC, num_subcores=NS)


def _deg_body(row2d, ones_hbm, zer_hbm, out_hbm, idx_v, ones_v, acc_sh):
    c = lax.axis_index("c")
    s = lax.axis_index("s")
    wid = c * NS + s
    pltpu.sync_copy(zer_hbm, acc_sh.at[pl.ds(s * RPT, RPT)])
    pltpu.sync_copy(ones_hbm, ones_v)
    pltpu.sync_copy(row2d.at[pl.ds(wid * NCH, NCH)], idx_v)
    plsc.subcore_barrier()

    def body(i, carry):
        pltpu.sync_copy(ones_v, acc_sh.at[idx_v.at[i]], add=True)
        return carry

    lax.fori_loop(0, NCH, body, 0)
    plsc.subcore_barrier()
    pltpu.sync_copy(acc_sh.at[pl.ds(s * RPT, RPT)],
                    out_hbm.at[pl.ds(c * NP + s * RPT, RPT)])


_sc_deg = pl.kernel(
    _deg_body,
    out_type=jax.ShapeDtypeStruct((2 * NP, 16), jnp.float32),
    mesh=_mesh,
    scratch_types=[
        pltpu.VMEM((NCH, CH), jnp.int32),
        pltpu.VMEM((CH, 16), jnp.float32),
        pltpu.VMEM_SHARED((NP, 16), jnp.float32),
    ],
    compiler_params=pltpu.CompilerParams(use_tc_tiling_on_sc=False),
)


def _disrow_body(dis16_hbm, row2d, out16, idx_v, buf, sem):
    c = lax.axis_index("c")
    s = lax.axis_index("s")
    wid = c * NS + s
    pltpu.sync_copy(row2d.at[pl.ds(wid * NCH, NCH)], idx_v)

    def body(i, carry):
        pltpu.async_copy(dis16_hbm.at[idx_v.at[i]], buf, sem).wait()
        pltpu.sync_copy(buf, out16.at[pl.ds(wid * EPT + i * CH, CH)])
        return carry

    lax.fori_loop(0, NCH, body, 0)


_sc_disrow = pl.kernel(
    _disrow_body,
    out_type=jax.ShapeDtypeStruct((EP, 16), jnp.float32),
    mesh=_mesh,
    scratch_types=[
        pltpu.VMEM((NCH, CH), jnp.int32),
        pltpu.VMEM((CH, 16), jnp.float32),
        pltpu.SemaphoreType.DMA,
    ],
    compiler_params=pltpu.CompilerParams(use_tc_tiling_on_sc=False),
)


DH = D // 2


def _gcn_body(xs_hbm, emb_hbm, row2d, col2d, zer_hbm, out_hbm,
              idxr, idxc, xg0, xg1, ev0, ev1, m0, m1, acc_sh,
              sg0, sg1, se0, se1, ss0, ss1):
    c = lax.axis_index("c")
    s = lax.axis_index("s")
    wid = c * NS + s

    def zbody(k, carry):
        pltpu.sync_copy(zer_hbm, acc_sh.at[pl.ds(s * RPT + k * CH, CH)])
        return carry

    lax.fori_loop(0, RPT // CH, zbody, 0)
    pltpu.sync_copy(row2d.at[pl.ds(wid * NCH, NCH)], idxr)
    pltpu.sync_copy(col2d.at[pl.ds(wid * NCH, NCH)], idxc)
    plsc.subcore_barrier()

    slots = ((xg0, ev0, m0, sg0, se0, ss0),
             (xg1, ev1, m1, sg1, se1, ss1))

    def issue_in(i, slot):
        xg, ev, _, sg, se, _ = slot
        pltpu.async_copy(xs_hbm.at[idxr.at[i]], xg, sg)
        pltpu.async_copy(emb_hbm.at[pl.ds(wid * EPT + i * CH, CH)], ev, se)

    def step(i, slot, wait_scatter):
        xg, ev, m, sg, se, ss = slot
        pltpu.make_async_copy(xs_hbm.at[idxr.at[i]], xg, sg).wait()
        pltpu.make_async_copy(
            emb_hbm.at[pl.ds(wid * EPT + i * CH, CH)], ev, se).wait()
        if wait_scatter:
            pltpu.make_async_copy(m, acc_sh.at[idxc.at[i]], ss).wait()

        def rowbody(r, cc):
            for j in range(DH // 16):
                sl = pl.ds(j * 16, 16)
                m[r, sl] = jnp.maximum(xg[r, sl] + ev[r, sl], 0.0)
            return cc

        lax.fori_loop(0, CH, rowbody, 0)
        pltpu.async_copy(m, acc_sh.at[idxc.at[i]], ss, add=True)

    issue_in(0, slots[0])
    issue_in(1, slots[1])
    step(0, slots[0], False)
    issue_in(2, slots[0])
    step(1, slots[1], False)
    issue_in(3, slots[1])

    def pair(k, carry):
        i0 = 2 * k
        step(i0, slots[0], True)

        @pl.when(i0 + 2 < NCH)
        def _():
            issue_in(i0 + 2, slots[0])

        step(i0 + 1, slots[1], True)

        @pl.when(i0 + 3 < NCH)
        def _():
            issue_in(i0 + 3, slots[1])

        return carry

    lax.fori_loop(1, NCH // 2, pair, 0)
    pltpu.make_async_copy(m0, acc_sh.at[idxc.at[0]], ss0).wait()
    pltpu.make_async_copy(m1, acc_sh.at[idxc.at[0]], ss1).wait()
    plsc.subcore_barrier()
    pltpu.sync_copy(acc_sh.at[pl.ds(s * RPT, RPT)],
                    out_hbm.at[pl.ds(c * NP + s * RPT, RPT)])


_sc_gcn = pl.kernel(
    _gcn_body,
    out_type=jax.ShapeDtypeStruct((2 * NP, DH), jnp.float32),
    mesh=_mesh,
    scratch_types=[
        pltpu.VMEM((NCH, CH), jnp.int32),
        pltpu.VMEM((NCH, CH), jnp.int32),
        pltpu.VMEM((CH, DH), jnp.float32),
        pltpu.VMEM((CH, DH), jnp.float32),
        pltpu.VMEM((CH, DH), jnp.float32),
        pltpu.VMEM((CH, DH), jnp.float32),
        pltpu.VMEM((CH, DH), jnp.float32),
        pltpu.VMEM((CH, DH), jnp.float32),
        pltpu.VMEM_SHARED((NP, DH), jnp.float32),
        pltpu.SemaphoreType.DMA,
        pltpu.SemaphoreType.DMA,
        pltpu.SemaphoreType.DMA,
        pltpu.SemaphoreType.DMA,
        pltpu.SemaphoreType.DMA,
        pltpu.SemaphoreType.DMA,
    ],
    compiler_params=pltpu.CompilerParams(use_tc_tiling_on_sc=False),
)


BNR = 1024
BER = 4096


def _enc_body(x_ref, wT_ref, b_ref, o_ref):
    o_ref[...] = jnp.maximum(
        jnp.dot(x_ref[...], wT_ref[...], preferred_element_type=jnp.float32)
        + b_ref[...], 0.0)


_tc_enc = pl.pallas_call(
    _enc_body,
    out_shape=jax.ShapeDtypeStruct((NP, D), jnp.float32),
    grid=(NP // BNR,),
    in_specs=[
        pl.BlockSpec((BNR, D), lambda i: (i, 0)),
        pl.BlockSpec((D, D), lambda i: (0, 0)),
        pl.BlockSpec((1, D), lambda i: (0, 0)),
    ],
    out_specs=pl.BlockSpec((BNR, D), lambda i: (i, 0)),
)


def _nodemm_body(h_ref, wT_ref, b_ref, dis_ref, y_ref, ys0_ref, ys1_ref):
    y = (jnp.dot(h_ref[...], wT_ref[...], preferred_element_type=jnp.float32)
         + b_ref[...])
    y_ref[...] = y
    ys = y * dis_ref[...]
    ys0_ref[...] = ys[:, :DH]
    ys1_ref[...] = ys[:, DH:]


_tc_nodemm = pl.pallas_call(
    _nodemm_body,
    out_shape=(jax.ShapeDtypeStruct((NP, D), jnp.float32),
               jax.ShapeDtypeStruct((NP, DH), jnp.float32),
               jax.ShapeDtypeStruct((NP, DH), jnp.float32)),
    grid=(NP // BNR,),
    in_specs=[
        pl.BlockSpec((BNR, D), lambda i: (i, 0)),
        pl.BlockSpec((D, D), lambda i: (0, 0)),
        pl.BlockSpec((1, D), lambda i: (0, 0)),
        pl.BlockSpec((BNR, 1), lambda i: (i, 0)),
    ],
    out_specs=(pl.BlockSpec((BNR, D), lambda i: (i, 0)),
               pl.BlockSpec((BNR, DH), lambda i: (i, 0)),
               pl.BlockSpec((BNR, DH), lambda i: (i, 0))),
)


def _edgemm_body(ea_ref, eWT_ref, eb_ref, dr_ref, o0_ref, o1_ref):
    m = (jnp.dot(ea_ref[...], eWT_ref[...], preferred_element_type=jnp.float32)
         + eb_ref[...]) * dr_ref[...]
    o0_ref[...] = m[:, :DH]
    o1_ref[...] = m[:, DH:]


_tc_edgemm = pl.pallas_call(
    _edgemm_body,
    out_shape=(jax.ShapeDtypeStruct((EP, DH), jnp.float32),
               jax.ShapeDtypeStruct((EP, DH), jnp.float32)),
    grid=(EP // BER,),
    in_specs=[
        pl.BlockSpec((BER, DE), lambda i: (i, 0)),
        pl.BlockSpec((DE, D), lambda i: (0, 0)),
        pl.BlockSpec((1, D), lambda i: (0, 0)),
        pl.BlockSpec((BER, 1), lambda i: (i, 0)),
    ],
    out_specs=(pl.BlockSpec((BER, DH), lambda i: (i, 0)),
               pl.BlockSpec((BER, DH), lambda i: (i, 0))),
)


def _prep_body(d0_ref, d1_ref, dis_ref, inv_ref, dis16_ref):
    deg = d0_ref[...] + d1_ref[...] + 1.0
    dis16 = lax.rsqrt(deg)
    dis16_ref[...] = dis16
    dis_ref[...] = dis16[:, :1]
    inv_ref[...] = 1.0 / deg[:, :1]


_tc_prep = pl.pallas_call(
    _prep_body,
    out_shape=(jax.ShapeDtypeStruct((NP, 1), jnp.float32),
               jax.ShapeDtypeStruct((NP, 1), jnp.float32),
               jax.ShapeDtypeStruct((NP, 16), jnp.float32)),
    grid=(NP // BNR,),
    in_specs=[
        pl.BlockSpec((BNR, 16), lambda i: (i, 0)),
        pl.BlockSpec((BNR, 16), lambda i: (i, 0)),
    ],
    out_specs=(pl.BlockSpec((BNR, 1), lambda i: (i, 0)),
               pl.BlockSpec((BNR, 1), lambda i: (i, 0)),
               pl.BlockSpec((BNR, 16), lambda i: (i, 0))),
)


def _make_combine(do_relu):
    def _combine_body(a0l_ref, a0h_ref, a1l_ref, a1h_ref, y_ref, dis_ref,
                      inv_ref, g_ref, bb_ref, o_ref):
        y = y_ref[...]
        acc = jnp.concatenate([a0l_ref[...] + a1l_ref[...],
                               a0h_ref[...] + a1h_ref[...]], axis=1)
        agg = dis_ref[...] * acc + jnp.maximum(y, 0.0) * inv_ref[...]
        r = agg * (g_ref[...] * BN_SCALE) + bb_ref[...]
        if do_relu:
            r = jnp.maximum(r, 0.0)
        o_ref[...] = r

    return pl.pallas_call(
        _combine_body,
        out_shape=jax.ShapeDtypeStruct((NP, D), jnp.float32),
        grid=(NP // BNR,),
        in_specs=[
            pl.BlockSpec((BNR, DH), lambda i: (i, 0)),
            pl.BlockSpec((BNR, DH), lambda i: (i, 0)),
            pl.BlockSpec((BNR, DH), lambda i: (i, 0)),
            pl.BlockSpec((BNR, DH), lambda i: (i, 0)),
            pl.BlockSpec((BNR, D), lambda i: (i, 0)),
            pl.BlockSpec((BNR, 1), lambda i: (i, 0)),
            pl.BlockSpec((BNR, 1), lambda i: (i, 0)),
            pl.BlockSpec((1, D), lambda i: (0, 0)),
            pl.BlockSpec((1, D), lambda i: (0, 0)),
        ],
        out_specs=pl.BlockSpec((BNR, D), lambda i: (i, 0)),
    )


_tc_combine_relu = _make_combine(True)
_tc_combine_last = _make_combine(False)


def kernel(x, edge_index, edge_attr, node_W, node_b,
           conv0_W, conv0_b, conv0_eW, conv0_eb, bn0_g, bn0_b,
           conv1_W, conv1_b, conv1_eW, conv1_eb, bn1_g, bn1_b,
           conv2_W, conv2_b, conv2_eW, conv2_eb, bn2_g, bn2_b):
    f32 = jnp.float32
    row = edge_index[0].astype(jnp.int32)
    col = edge_index[1].astype(jnp.int32)
    pad_idx = jnp.full((EP - E,), N, jnp.int32)
    row2d = jnp.concatenate([row, pad_idx]).reshape(ROWS2D, CH)
    col2d = jnp.concatenate([col, pad_idx]).reshape(ROWS2D, CH)
    eap = jnp.pad(edge_attr.astype(f32), ((0, EP - E), (0, 0)))
    xp = jnp.pad(x.astype(f32), ((0, NP - N), (0, 0)))

    ones16 = jnp.ones((CH, 16), f32)
    zer16 = jnp.zeros((RPT, 16), f32)
    zerD = jnp.zeros((CH, DH), f32)

    deg2 = _sc_deg(row2d, ones16, zer16)
    dis, inv, dis16 = _tc_prep(deg2[:NP], deg2[NP:])
    dis_row = _sc_disrow(dis16, row2d)[:, :1]

    h = _tc_enc(xp, node_W.T, node_b.reshape(1, D))

    convs = [
        (conv0_W, conv0_b, conv0_eW, conv0_eb, bn0_g, bn0_b, _tc_combine_relu),
        (conv1_W, conv1_b, conv1_eW, conv1_eb, bn1_g, bn1_b, _tc_combine_relu),
        (conv2_W, conv2_b, conv2_eW, conv2_eb, bn2_g, bn2_b, _tc_combine_last),
    ]
    for W, b, eW, eb, g, bb, combine in convs:
        y, ys0, ys1 = _tc_nodemm(h, W.T, b.reshape(1, D), dis)
        emb0, emb1 = _tc_edgemm(eap, eW.T, eb.reshape(1, D), dis_row)
        acc_lo = _sc_gcn(ys0, emb0, row2d, col2d, zerD)
        acc_hi = _sc_gcn(ys1, emb1, row2d, col2d, zerD)
        h = combine(acc_lo[:NP], acc_hi[:NP], acc_lo[NP:], acc_hi[NP:],
                    y, dis, inv, g.reshape(1, D), bb.reshape(1, D))

    return h[:N]


# speedup vs baseline: 6.6133x; 2.2218x over previous
import math

import jax
import jax.numpy as jnp
from jax import lax
from jax.experimental import pallas as pl
from jax.experimental.pallas import tpu as pltpu
from jax.experimental.pallas import tpu_sc as plsc

N = 10000
D = 128
DH = D // 2
DE = 16
E = 320000
EPS = 1e-5

NC = 2
NS = 16
NW = NC * NS
CH = 125
ROWS2D = E // CH
EPT = E // NW
NCH = EPT // CH
EPTM = E // NS
NCHM = EPTM // CH
RPT = N // NS

BN_SCALE = 1.0 / math.sqrt(1.0 + EPS)

_mesh = plsc.VectorSubcoreMesh(core_axis_name="c", subcore_axis_name="s",
                               num_cores=NC, num_subcores=NS)
_sc_params = pltpu.CompilerParams(use_tc_tiling_on_sc=False)


def _deg_body(row2d, ones_hbm, zer_hbm, out_hbm, idx_v, ones_v, acc_sh):
    c = lax.axis_index("c")
    s = lax.axis_index("s")
    wid = c * NS + s

    def zbody(k, carry):
        pltpu.sync_copy(zer_hbm, acc_sh.at[pl.ds(s * RPT + k * CH, CH)])
        return carry

    lax.fori_loop(0, RPT // CH, zbody, 0)
    pltpu.sync_copy(ones_hbm, ones_v)
    pltpu.sync_copy(row2d.at[pl.ds(wid * NCH, NCH)], idx_v)
    plsc.subcore_barrier()

    def body(i, carry):
        pltpu.sync_copy(ones_v, acc_sh.at[idx_v.at[i]], add=True)
        return carry

    lax.fori_loop(0, NCH, body, 0)
    plsc.subcore_barrier()
    pltpu.sync_copy(acc_sh.at[pl.ds(s * RPT, RPT)],
                    out_hbm.at[pl.ds(c * N + s * RPT, RPT)])


_sc_deg = pl.kernel(
    _deg_body,
    out_type=jax.ShapeDtypeStruct((2 * N, 16), jnp.float32),
    mesh=_mesh,
    scratch_types=[
        pltpu.VMEM((NCH, CH), jnp.int32),
        pltpu.VMEM((CH, 16), jnp.float32),
        pltpu.VMEM_SHARED((N, 16), jnp.float32),
    ],
    compiler_params=_sc_params,
)


def _disrow_body(dis16_hbm, row2d, out16, idx_v, buf, sem):
    c = lax.axis_index("c")
    s = lax.axis_index("s")
    wid = c * NS + s
    pltpu.sync_copy(row2d.at[pl.ds(wid * NCH, NCH)], idx_v)

    def body(i, carry):
        pltpu.async_copy(dis16_hbm.at[idx_v.at[i]], buf, sem).wait()
        pltpu.sync_copy(buf, out16.at[pl.ds(wid * EPT + i * CH, CH)])
        return carry

    lax.fori_loop(0, NCH, body, 0)


_sc_disrow = pl.kernel(
    _disrow_body,
    out_type=jax.ShapeDtypeStruct((E, 16), jnp.float32),
    mesh=_mesh,
    scratch_types=[
        pltpu.VMEM((NCH, CH), jnp.int32),
        pltpu.VMEM((CH, 16), jnp.float32),
        pltpu.SemaphoreType.DMA,
    ],
    compiler_params=_sc_params,
)


def _make_gcn(hoff):
    def _gcn_body(xs_hbm, emb_hbm, row2d, col2d, zer_hbm, out_hbm,
                  idxr, idxc, xg0, xg1, ev0, ev1, m0, m1, acc_sh,
                  sg0, sg1, se0, se1, ss0, ss1):
        c = lax.axis_index("c")
        s = lax.axis_index("s")
        wid = c * NS + s

        def zbody(k, carry):
            pltpu.sync_copy(zer_hbm, acc_sh.at[pl.ds(s * RPT + k * CH, CH)])
            return carry

        lax.fori_loop(0, RPT // CH, zbody, 0)
        pltpu.sync_copy(row2d.at[pl.ds(wid * NCH, NCH)], idxr)
        pltpu.sync_copy(col2d.at[pl.ds(wid * NCH, NCH)], idxc)
        plsc.subcore_barrier()

        slots = ((xg0, ev0, m0, sg0, se0, ss0),
                 (xg1, ev1, m1, sg1, se1, ss1))

        def issue_in(i, slot):
            xg, ev, _, sg, se, _ = slot
            pltpu.async_copy(xs_hbm.at[idxr.at[i]], xg, sg)
            pltpu.async_copy(
                emb_hbm.at[pl.ds(wid * EPT + i * CH, CH), pl.ds(hoff, DH)],
                ev, se)

        def step(i, slot, wait_scatter):
            xg, ev, m, sg, se, ss = slot
            pltpu.make_async_copy(xs_hbm.at[idxr.at[i]], xg, sg).wait()
            pltpu.make_async_copy(
                emb_hbm.at[pl.ds(wid * EPT + i * CH, CH), pl.ds(hoff, DH)],
                ev, se).wait()
            if wait_scatter:
                pltpu.make_async_copy(m, acc_sh.at[idxc.at[i]], ss).wait()

            def rowbody(r, cc):
                for j in range(DH // 16):
                    sl = pl.ds(j * 16, 16)
                    m[r, sl] = jnp.maximum(xg[r, sl] + ev[r, sl], 0.0)
                return cc

            lax.fori_loop(0, CH, rowbody, 0)
            pltpu.async_copy(m, acc_sh.at[idxc.at[i]], ss, add=True)

        issue_in(0, slots[0])
        issue_in(1, slots[1])
        step(0, slots[0], False)
        issue_in(2, slots[0])
        step(1, slots[1], False)
        issue_in(3, slots[1])

        def pair(k, carry):
            i0 = 2 * k
            step(i0, slots[0], True)

            @pl.when(i0 + 2 < NCH)
            def _():
                issue_in(i0 + 2, slots[0])

            step(i0 + 1, slots[1], True)

            @pl.when(i0 + 3 < NCH)
            def _():
                issue_in(i0 + 3, slots[1])

            return carry

        lax.fori_loop(1, NCH // 2, pair, 0)
        pltpu.make_async_copy(m0, acc_sh.at[idxc.at[0]], ss0).wait()
        pltpu.make_async_copy(m1, acc_sh.at[idxc.at[0]], ss1).wait()
        plsc.subcore_barrier()
        pltpu.sync_copy(acc_sh.at[pl.ds(s * RPT, RPT)],
                        out_hbm.at[pl.ds(c * N + s * RPT, RPT)])

    return pl.kernel(
        _gcn_body,
        out_type=jax.ShapeDtypeStruct((2 * N, DH), jnp.float32),
        mesh=_mesh,
        scratch_types=[
            pltpu.VMEM((NCH, CH), jnp.int32),
            pltpu.VMEM((NCH, CH), jnp.int32),
            pltpu.VMEM((CH, DH), jnp.float32),
            pltpu.VMEM((CH, DH), jnp.float32),
            pltpu.VMEM((CH, DH), jnp.float32),
            pltpu.VMEM((CH, DH), jnp.float32),
            pltpu.VMEM((CH, DH), jnp.float32),
            pltpu.VMEM((CH, DH), jnp.float32),
            pltpu.VMEM_SHARED((N, DH), jnp.float32),
            pltpu.SemaphoreType.DMA,
            pltpu.SemaphoreType.DMA,
            pltpu.SemaphoreType.DMA,
            pltpu.SemaphoreType.DMA,
            pltpu.SemaphoreType.DMA,
            pltpu.SemaphoreType.DMA,
        ],
        compiler_params=_sc_params,
    )


_sc_gcn_lo = _make_gcn(0)
_sc_gcn_hi = _make_gcn(DH)


BNR = 1000
BER = 4000


def _enc_body(x_ref, wT_ref, b_ref, o_ref):
    o_ref[...] = jnp.maximum(
        jnp.dot(x_ref[...], wT_ref[...], preferred_element_type=jnp.float32)
        + b_ref[...], 0.0)


_tc_enc = pl.pallas_call(
    _enc_body,
    out_shape=jax.ShapeDtypeStruct((N, D), jnp.float32),
    grid=(N // BNR,),
    in_specs=[
        pl.BlockSpec((BNR, D), lambda i: (i, 0)),
        pl.BlockSpec((D, D), lambda i: (0, 0)),
        pl.BlockSpec((1, D), lambda i: (0, 0)),
    ],
    out_specs=pl.BlockSpec((BNR, D), lambda i: (i, 0)),
)


def _nodemm_body(h_ref, wT_ref, b_ref, dis_ref, y_ref, ys0_ref, ys1_ref):
    y = (jnp.dot(h_ref[...], wT_ref[...], preferred_element_type=jnp.float32)
         + b_ref[...])
    y_ref[...] = y
    ys = y * dis_ref[...]
    ys0_ref[...] = ys[:, :DH]
    ys1_ref[...] = ys[:, DH:]


_tc_nodemm = pl.pallas_call(
    _nodemm_body,
    out_shape=(jax.ShapeDtypeStruct((N, D), jnp.float32),
               jax.ShapeDtypeStruct((N, DH), jnp.float32),
               jax.ShapeDtypeStruct((N, DH), jnp.float32)),
    grid=(N // BNR,),
    in_specs=[
        pl.BlockSpec((BNR, D), lambda i: (i, 0)),
        pl.BlockSpec((D, D), lambda i: (0, 0)),
        pl.BlockSpec((1, D), lambda i: (0, 0)),
        pl.BlockSpec((BNR, 1), lambda i: (i, 0)),
    ],
    out_specs=(pl.BlockSpec((BNR, D), lambda i: (i, 0)),
               pl.BlockSpec((BNR, DH), lambda i: (i, 0)),
               pl.BlockSpec((BNR, DH), lambda i: (i, 0))),
)


def _edgemm_body(ea_ref, eWT_ref, eb_ref, dr_ref, o_ref):
    o_ref[...] = (jnp.dot(ea_ref[...], eWT_ref[...],
                          preferred_element_type=jnp.float32)
                  + eb_ref[...]) * dr_ref[...]


_tc_edgemm = pl.pallas_call(
    _edgemm_body,
    out_shape=jax.ShapeDtypeStruct((E, D), jnp.float32),
    grid=(E // BER,),
    in_specs=[
        pl.BlockSpec((BER, DE), lambda i: (i, 0)),
        pl.BlockSpec((DE, D), lambda i: (0, 0)),
        pl.BlockSpec((1, D), lambda i: (0, 0)),
        pl.BlockSpec((BER, 1), lambda i: (i, 0)),
    ],
    out_specs=pl.BlockSpec((BER, D), lambda i: (i, 0)),
)


def _prep_body(d0_ref, d1_ref, dis_ref, inv_ref, dis16_ref):
    deg = d0_ref[...] + d1_ref[...] + 1.0
    dis16 = lax.rsqrt(deg)
    dis16_ref[...] = dis16
    dis_ref[...] = dis16[:, :1]
    inv_ref[...] = 1.0 / deg[:, :1]


_tc_prep = pl.pallas_call(
    _prep_body,
    out_shape=(jax.ShapeDtypeStruct((N, 1), jnp.float32),
               jax.ShapeDtypeStruct((N, 1), jnp.float32),
               jax.ShapeDtypeStruct((N, 16), jnp.float32)),
    grid=(N // BNR,),
    in_specs=[
        pl.BlockSpec((BNR, 16), lambda i: (i, 0)),
        pl.BlockSpec((BNR, 16), lambda i: (i, 0)),
    ],
    out_specs=(pl.BlockSpec((BNR, 1), lambda i: (i, 0)),
               pl.BlockSpec((BNR, 1), lambda i: (i, 0)),
               pl.BlockSpec((BNR, 16), lambda i: (i, 0))),
)


def _make_combine(do_relu):
    def _combine_body(a0l_ref, a1l_ref, a0h_ref, a1h_ref, y_ref, dis_ref,
                      inv_ref, g_ref, bb_ref, o_ref):
        y = y_ref[...]
        acc = jnp.concatenate([a0l_ref[...] + a1l_ref[...],
                               a0h_ref[...] + a1h_ref[...]], axis=1)
        agg = dis_ref[...] * acc + jnp.maximum(y, 0.0) * inv_ref[...]
        r = agg * (g_ref[...] * BN_SCALE) + bb_ref[...]
        if do_relu:
            r = jnp.maximum(r, 0.0)
        o_ref[...] = r

    return pl.pallas_call(
        _combine_body,
        out_shape=jax.ShapeDtypeStruct((N, D), jnp.float32),
        grid=(N // BNR,),
        in_specs=[
            pl.BlockSpec((BNR, DH), lambda i: (i, 0)),
            pl.BlockSpec((BNR, DH), lambda i: (i, 0)),
            pl.BlockSpec((BNR, DH), lambda i: (i, 0)),
            pl.BlockSpec((BNR, DH), lambda i: (i, 0)),
            pl.BlockSpec((BNR, D), lambda i: (i, 0)),
            pl.BlockSpec((BNR, 1), lambda i: (i, 0)),
            pl.BlockSpec((BNR, 1), lambda i: (i, 0)),
            pl.BlockSpec((1, D), lambda i: (0, 0)),
            pl.BlockSpec((1, D), lambda i: (0, 0)),
        ],
        out_specs=pl.BlockSpec((BNR, D), lambda i: (i, 0)),
    )


_tc_combine_relu = _make_combine(True)
_tc_combine_last = _make_combine(False)


def kernel(x, edge_index, edge_attr, node_W, node_b,
           conv0_W, conv0_b, conv0_eW, conv0_eb, bn0_g, bn0_b,
           conv1_W, conv1_b, conv1_eW, conv1_eb, bn1_g, bn1_b,
           conv2_W, conv2_b, conv2_eW, conv2_eb, bn2_g, bn2_b):
    f32 = jnp.float32
    row2d = edge_index[0].astype(jnp.int32).reshape(ROWS2D, CH)
    col2d = edge_index[1].astype(jnp.int32).reshape(ROWS2D, CH)
    ea = edge_attr.astype(f32)
    xf = x.astype(f32)

    ones16 = jnp.ones((CH, 16), f32)
    zer16 = jnp.zeros((CH, 16), f32)
    zerD = jnp.zeros((CH, DH), f32)

    deg2 = _sc_deg(row2d, ones16, zer16)
    dis, inv, dis16 = _tc_prep(deg2[:N], deg2[N:])
    dis_row = _sc_disrow(dis16, row2d)[:, :1]

    h = _tc_enc(xf, node_W.T, node_b.reshape(1, D))

    convs = [
        (conv0_W, conv0_b, conv0_eW, conv0_eb, bn0_g, bn0_b, _tc_combine_relu),
        (conv1_W, conv1_b, conv1_eW, conv1_eb, bn1_g, bn1_b, _tc_combine_relu),
        (conv2_W, conv2_b, conv2_eW, conv2_eb, bn2_g, bn2_b, _tc_combine_last),
    ]
    for W, b, eW, eb, g, bb, combine in convs:
        y, ys0, ys1 = _tc_nodemm(h, W.T, b.reshape(1, D), dis)
        emb = _tc_edgemm(ea, eW.T, eb.reshape(1, D), dis_row)
        acc_lo = _sc_gcn_lo(ys0, emb, row2d, col2d, zerD)
        acc_hi = _sc_gcn_hi(ys1, emb, row2d, col2d, zerD)
        h = combine(acc_lo[:N], acc_lo[N:], acc_hi[:N], acc_hi[N:],
                    y, dis, inv, g.reshape(1, D), bb.reshape(1, D))

    return h


# speedup vs baseline: 6.7102x; 1.0146x over previous
import math

import jax
import jax.numpy as jnp
from jax import lax
from jax.experimental import pallas as pl
from jax.experimental.pallas import tpu as pltpu
from jax.experimental.pallas import tpu_sc as plsc

N = 10000
D = 128
DH = D // 2
DE = 16
E = 320000
EPS = 1e-5

NC = 2
NS = 16
NW = NC * NS
CH = 125
ROWS2D = E // CH
EPT = E // NW
NCH = EPT // CH
EPTM = E // NS
NCHM = EPTM // CH
RPT = N // NS

BN_SCALE = 1.0 / math.sqrt(1.0 + EPS)

_mesh = plsc.VectorSubcoreMesh(core_axis_name="c", subcore_axis_name="s",
                               num_cores=NC, num_subcores=NS)
_sc_params = pltpu.CompilerParams(use_tc_tiling_on_sc=False)


def _deg_body(row2d, ones_hbm, zer_hbm, out_hbm, idx_v, ones_v, acc_sh):
    c = lax.axis_index("c")
    s = lax.axis_index("s")
    wid = c * NS + s

    def zbody(k, carry):
        pltpu.sync_copy(zer_hbm, acc_sh.at[pl.ds(s * RPT + k * CH, CH)])
        return carry

    lax.fori_loop(0, RPT // CH, zbody, 0)
    pltpu.sync_copy(ones_hbm, ones_v)
    pltpu.sync_copy(row2d.at[pl.ds(wid * NCH, NCH)], idx_v)
    plsc.subcore_barrier()

    def body(i, carry):
        pltpu.sync_copy(ones_v, acc_sh.at[idx_v.at[i]], add=True)
        return carry

    lax.fori_loop(0, NCH, body, 0)
    plsc.subcore_barrier()
    pltpu.sync_copy(acc_sh.at[pl.ds(s * RPT, RPT)],
                    out_hbm.at[pl.ds(c * N + s * RPT, RPT)])


_sc_deg = pl.kernel(
    _deg_body,
    out_type=jax.ShapeDtypeStruct((2 * N, 16), jnp.float32),
    mesh=_mesh,
    scratch_types=[
        pltpu.VMEM((NCH, CH), jnp.int32),
        pltpu.VMEM((CH, 16), jnp.float32),
        pltpu.VMEM_SHARED((N, 16), jnp.float32),
    ],
    compiler_params=_sc_params,
)


def _disrow_body(dis16_hbm, row2d, out16, idx_v, buf, sem):
    c = lax.axis_index("c")
    s = lax.axis_index("s")
    wid = c * NS + s
    pltpu.sync_copy(row2d.at[pl.ds(wid * NCH, NCH)], idx_v)

    def body(i, carry):
        pltpu.async_copy(dis16_hbm.at[idx_v.at[i]], buf, sem).wait()
        pltpu.sync_copy(buf, out16.at[pl.ds(wid * EPT + i * CH, CH)])
        return carry

    lax.fori_loop(0, NCH, body, 0)


_sc_disrow = pl.kernel(
    _disrow_body,
    out_type=jax.ShapeDtypeStruct((E, 16), jnp.float32),
    mesh=_mesh,
    scratch_types=[
        pltpu.VMEM((NCH, CH), jnp.int32),
        pltpu.VMEM((CH, 16), jnp.float32),
        pltpu.SemaphoreType.DMA,
    ],
    compiler_params=_sc_params,
)


def _make_gcn(hoff):
    def _gcn_body(xs_hbm, emb_hbm, row2d, col2d, zer_hbm, out_hbm,
                  idxr, idxc, xg0, xg1, ev0, ev1, m0, m1, acc_sh,
                  sg0, sg1, se0, se1, ss0, ss1):
        c = lax.axis_index("c")
        s = lax.axis_index("s")
        wid = c * NS + s

        def zbody(k, carry):
            pltpu.sync_copy(zer_hbm, acc_sh.at[pl.ds(s * RPT + k * CH, CH)])
            return carry

        lax.fori_loop(0, RPT // CH, zbody, 0)
        pltpu.sync_copy(row2d.at[pl.ds(wid * NCH, NCH)], idxr)
        pltpu.sync_copy(col2d.at[pl.ds(wid * NCH, NCH)], idxc)
        plsc.subcore_barrier()

        slots = ((xg0, ev0, m0, sg0, se0, ss0),
                 (xg1, ev1, m1, sg1, se1, ss1))

        def issue_in(i, slot):
            xg, ev, _, sg, se, _ = slot
            pltpu.async_copy(xs_hbm.at[idxr.at[i]], xg, sg)
            pltpu.async_copy(
                emb_hbm.at[pl.ds(wid * EPT + i * CH, CH), pl.ds(hoff, DH)],
                ev, se)

        def step(i, slot, wait_scatter):
            xg, ev, m, sg, se, ss = slot
            pltpu.make_async_copy(xs_hbm.at[idxr.at[i]], xg, sg).wait()
            pltpu.make_async_copy(
                emb_hbm.at[pl.ds(wid * EPT + i * CH, CH), pl.ds(hoff, DH)],
                ev, se).wait()
            if wait_scatter:
                pltpu.make_async_copy(m, acc_sh.at[idxc.at[i]], ss).wait()

            def rowbody(r, cc):
                for j in range(DH // 16):
                    sl = pl.ds(j * 16, 16)
                    m[r, sl] = jnp.maximum(xg[r, sl] + ev[r, sl], 0.0)
                return cc

            lax.fori_loop(0, CH, rowbody, 0)
            pltpu.async_copy(m, acc_sh.at[idxc.at[i]], ss, add=True)

        issue_in(0, slots[0])
        issue_in(1, slots[1])
        step(0, slots[0], False)
        issue_in(2, slots[0])
        step(1, slots[1], False)
        issue_in(3, slots[1])

        def pair(k, carry):
            i0 = 2 * k
            step(i0, slots[0], True)

            @pl.when(i0 + 2 < NCH)
            def _():
                issue_in(i0 + 2, slots[0])

            step(i0 + 1, slots[1], True)

            @pl.when(i0 + 3 < NCH)
            def _():
                issue_in(i0 + 3, slots[1])

            return carry

        lax.fori_loop(1, NCH // 2, pair, 0)
        pltpu.make_async_copy(m0, acc_sh.at[idxc.at[0]], ss0).wait()
        pltpu.make_async_copy(m1, acc_sh.at[idxc.at[0]], ss1).wait()
        plsc.subcore_barrier()
        pltpu.sync_copy(acc_sh.at[pl.ds(s * RPT, RPT)],
                        out_hbm.at[pl.ds(c * N + s * RPT, RPT)])

    return pl.kernel(
        _gcn_body,
        out_type=jax.ShapeDtypeStruct((2 * N, DH), jnp.float32),
        mesh=_mesh,
        scratch_types=[
            pltpu.VMEM((NCH, CH), jnp.int32),
            pltpu.VMEM((NCH, CH), jnp.int32),
            pltpu.VMEM((CH, DH), jnp.float32),
            pltpu.VMEM((CH, DH), jnp.float32),
            pltpu.VMEM((CH, DH), jnp.float32),
            pltpu.VMEM((CH, DH), jnp.float32),
            pltpu.VMEM((CH, DH), jnp.float32),
            pltpu.VMEM((CH, DH), jnp.float32),
            pltpu.VMEM_SHARED((N, DH), jnp.float32),
            pltpu.SemaphoreType.DMA,
            pltpu.SemaphoreType.DMA,
            pltpu.SemaphoreType.DMA,
            pltpu.SemaphoreType.DMA,
            pltpu.SemaphoreType.DMA,
            pltpu.SemaphoreType.DMA,
        ],
        compiler_params=_sc_params,
    )


_sc_gcn_lo = _make_gcn(0)
_sc_gcn_hi = _make_gcn(DH)


BNR = 1000
BER = 4000


def _enc_nodemm_body(x_ref, nwT_ref, nb_ref, wT_ref, b_ref, dis_ref,
                     y_ref, ys0_ref, ys1_ref):
    h = jnp.maximum(
        jnp.dot(x_ref[...], nwT_ref[...], preferred_element_type=jnp.float32)
        + nb_ref[...], 0.0)
    y = (jnp.dot(h, wT_ref[...], preferred_element_type=jnp.float32)
         + b_ref[...])
    y_ref[...] = y
    ys = y * dis_ref[...]
    ys0_ref[...] = ys[:, :DH]
    ys1_ref[...] = ys[:, DH:]


_tc_enc_nodemm = pl.pallas_call(
    _enc_nodemm_body,
    out_shape=(jax.ShapeDtypeStruct((N, D), jnp.float32),
               jax.ShapeDtypeStruct((N, DH), jnp.float32),
               jax.ShapeDtypeStruct((N, DH), jnp.float32)),
    grid=(N // BNR,),
    in_specs=[
        pl.BlockSpec((BNR, D), lambda i: (i, 0)),
        pl.BlockSpec((D, D), lambda i: (0, 0)),
        pl.BlockSpec((1, D), lambda i: (0, 0)),
        pl.BlockSpec((D, D), lambda i: (0, 0)),
        pl.BlockSpec((1, D), lambda i: (0, 0)),
        pl.BlockSpec((BNR, 1), lambda i: (i, 0)),
    ],
    out_specs=(pl.BlockSpec((BNR, D), lambda i: (i, 0)),
               pl.BlockSpec((BNR, DH), lambda i: (i, 0)),
               pl.BlockSpec((BNR, DH), lambda i: (i, 0))),
)


def _combine_expr(a0l, a1l, a0h, a1h, y, dis, inv, g, bb):
    acc = jnp.concatenate([a0l + a1l, a0h + a1h], axis=1)
    agg = dis * acc + jnp.maximum(y, 0.0) * inv
    return agg * (g * BN_SCALE) + bb


def _combine_nodemm_body(a0l_ref, a1l_ref, a0h_ref, a1h_ref, y_ref, dis_ref,
                         inv_ref, g_ref, bb_ref, wT_ref, b_ref,
                         yn_ref, ys0_ref, ys1_ref):
    h = jnp.maximum(
        _combine_expr(a0l_ref[...], a1l_ref[...], a0h_ref[...], a1h_ref[...],
                      y_ref[...], dis_ref[...], inv_ref[...], g_ref[...],
                      bb_ref[...]), 0.0)
    y = (jnp.dot(h, wT_ref[...], preferred_element_type=jnp.float32)
         + b_ref[...])
    yn_ref[...] = y
    ys = y * dis_ref[...]
    ys0_ref[...] = ys[:, :DH]
    ys1_ref[...] = ys[:, DH:]


_tc_combine_nodemm = pl.pallas_call(
    _combine_nodemm_body,
    out_shape=(jax.ShapeDtypeStruct((N, D), jnp.float32),
               jax.ShapeDtypeStruct((N, DH), jnp.float32),
               jax.ShapeDtypeStruct((N, DH), jnp.float32)),
    grid=(N // BNR,),
    in_specs=[
        pl.BlockSpec((BNR, DH), lambda i: (i, 0)),
        pl.BlockSpec((BNR, DH), lambda i: (i, 0)),
        pl.BlockSpec((BNR, DH), lambda i: (i, 0)),
        pl.BlockSpec((BNR, DH), lambda i: (i, 0)),
        pl.BlockSpec((BNR, D), lambda i: (i, 0)),
        pl.BlockSpec((BNR, 1), lambda i: (i, 0)),
        pl.BlockSpec((BNR, 1), lambda i: (i, 0)),
        pl.BlockSpec((1, D), lambda i: (0, 0)),
        pl.BlockSpec((1, D), lambda i: (0, 0)),
        pl.BlockSpec((D, D), lambda i: (0, 0)),
        pl.BlockSpec((1, D), lambda i: (0, 0)),
    ],
    out_specs=(pl.BlockSpec((BNR, D), lambda i: (i, 0)),
               pl.BlockSpec((BNR, DH), lambda i: (i, 0)),
               pl.BlockSpec((BNR, DH), lambda i: (i, 0))),
)


def _edgemm_body(ea_ref, eWT_ref, eb_ref, dr_ref, o_ref):
    o_ref[...] = (jnp.dot(ea_ref[...], eWT_ref[...],
                          preferred_element_type=jnp.float32)
                  + eb_ref[...]) * dr_ref[...]


_tc_edgemm = pl.pallas_call(
    _edgemm_body,
    out_shape=jax.ShapeDtypeStruct((E, D), jnp.float32),
    grid=(E // BER,),
    in_specs=[
        pl.BlockSpec((BER, DE), lambda i: (i, 0)),
        pl.BlockSpec((DE, D), lambda i: (0, 0)),
        pl.BlockSpec((1, D), lambda i: (0, 0)),
        pl.BlockSpec((BER, 1), lambda i: (i, 0)),
    ],
    out_specs=pl.BlockSpec((BER, D), lambda i: (i, 0)),
)


def _prep_body(d0_ref, d1_ref, dis_ref, inv_ref, dis16_ref):
    deg = d0_ref[...] + d1_ref[...] + 1.0
    dis16 = lax.rsqrt(deg)
    dis16_ref[...] = dis16
    dis_ref[...] = dis16[:, :1]
    inv_ref[...] = 1.0 / deg[:, :1]


_tc_prep = pl.pallas_call(
    _prep_body,
    out_shape=(jax.ShapeDtypeStruct((N, 1), jnp.float32),
               jax.ShapeDtypeStruct((N, 1), jnp.float32),
               jax.ShapeDtypeStruct((N, 16), jnp.float32)),
    grid=(N // BNR,),
    in_specs=[
        pl.BlockSpec((BNR, 16), lambda i: (i, 0)),
        pl.BlockSpec((BNR, 16), lambda i: (i, 0)),
    ],
    out_specs=(pl.BlockSpec((BNR, 1), lambda i: (i, 0)),
               pl.BlockSpec((BNR, 1), lambda i: (i, 0)),
               pl.BlockSpec((BNR, 16), lambda i: (i, 0))),
)


def _combine_last_body(a0l_ref, a1l_ref, a0h_ref, a1h_ref, y_ref, dis_ref,
                       inv_ref, g_ref, bb_ref, o_ref):
    o_ref[...] = _combine_expr(
        a0l_ref[...], a1l_ref[...], a0h_ref[...], a1h_ref[...], y_ref[...],
        dis_ref[...], inv_ref[...], g_ref[...], bb_ref[...])


_tc_combine_last = pl.pallas_call(
    _combine_last_body,
    out_shape=jax.ShapeDtypeStruct((N, D), jnp.float32),
    grid=(N // BNR,),
    in_specs=[
        pl.BlockSpec((BNR, DH), lambda i: (i, 0)),
        pl.BlockSpec((BNR, DH), lambda i: (i, 0)),
        pl.BlockSpec((BNR, DH), lambda i: (i, 0)),
        pl.BlockSpec((BNR, DH), lambda i: (i, 0)),
        pl.BlockSpec((BNR, D), lambda i: (i, 0)),
        pl.BlockSpec((BNR, 1), lambda i: (i, 0)),
        pl.BlockSpec((BNR, 1), lambda i: (i, 0)),
        pl.BlockSpec((1, D), lambda i: (0, 0)),
        pl.BlockSpec((1, D), lambda i: (0, 0)),
    ],
    out_specs=pl.BlockSpec((BNR, D), lambda i: (i, 0)),
)


def kernel(x, edge_index, edge_attr, node_W, node_b,
           conv0_W, conv0_b, conv0_eW, conv0_eb, bn0_g, bn0_b,
           conv1_W, conv1_b, conv1_eW, conv1_eb, bn1_g, bn1_b,
           conv2_W, conv2_b, conv2_eW, conv2_eb, bn2_g, bn2_b):
    f32 = jnp.float32
    row2d = edge_index[0].astype(jnp.int32).reshape(ROWS2D, CH)
    col2d = edge_index[1].astype(jnp.int32).reshape(ROWS2D, CH)
    ea = edge_attr.astype(f32)
    xf = x.astype(f32)

    ones16 = jnp.ones((CH, 16), f32)
    zer16 = jnp.zeros((CH, 16), f32)
    zerD = jnp.zeros((CH, DH), f32)

    deg2 = _sc_deg(row2d, ones16, zer16)
    dis, inv, dis16 = _tc_prep(deg2[:N], deg2[N:])
    dis_row = _sc_disrow(dis16, row2d)[:, :1]

    y, ys0, ys1 = _tc_enc_nodemm(xf, node_W.T, node_b.reshape(1, D),
                                 conv0_W.T, conv0_b.reshape(1, D), dis)

    convs = [
        (conv0_eW, conv0_eb, bn0_g, bn0_b, conv1_W, conv1_b),
        (conv1_eW, conv1_eb, bn1_g, bn1_b, conv2_W, conv2_b),
        (conv2_eW, conv2_eb, bn2_g, bn2_b, None, None),
    ]
    for eW, eb, g, bb, Wn, bn in convs:
        emb = _tc_edgemm(ea, eW.T, eb.reshape(1, D), dis_row)
        acc_lo = _sc_gcn_lo(ys0, emb, row2d, col2d, zerD)
        acc_hi = _sc_gcn_hi(ys1, emb, row2d, col2d, zerD)
        if Wn is not None:
            y, ys0, ys1 = _tc_combine_nodemm(
                acc_lo[:N], acc_lo[N:], acc_hi[:N], acc_hi[N:], y, dis, inv,
                g.reshape(1, D), bb.reshape(1, D), Wn.T, bn.reshape(1, D))
        else:
            h = _tc_combine_last(
                acc_lo[:N], acc_lo[N:], acc_hi[:N], acc_hi[N:], y, dis, inv,
                g.reshape(1, D), bb.reshape(1, D))

    return h


# speedup vs baseline: 7.5989x; 1.1324x over previous
import math

import jax
import jax.numpy as jnp
from jax import lax
from jax.experimental import pallas as pl
from jax.experimental.pallas import tpu as pltpu
from jax.experimental.pallas import tpu_sc as plsc

N = 10000
D = 128
DH = D // 2
DE = 16
E = 320000
EPS = 1e-5

NC = 2
NS = 16
NW = NC * NS
CH = 125
ROWS2D = E // CH
EPT = E // NW
NCH = EPT // CH
EPTM = E // NS
NCHM = EPTM // CH
RPT = N // NS

BN_SCALE = 1.0 / math.sqrt(1.0 + EPS)

_mesh = plsc.VectorSubcoreMesh(core_axis_name="c", subcore_axis_name="s",
                               num_cores=NC, num_subcores=NS)
_sc_params = pltpu.CompilerParams(use_tc_tiling_on_sc=False)


def _deg_body(row2d, ones_hbm, zer_hbm, out_hbm, idx_v, ones_v, acc_sh):
    c = lax.axis_index("c")
    s = lax.axis_index("s")
    wid = c * NS + s

    def zbody(k, carry):
        pltpu.sync_copy(zer_hbm, acc_sh.at[pl.ds(s * RPT + k * CH, CH)])
        return carry

    lax.fori_loop(0, RPT // CH, zbody, 0)
    pltpu.sync_copy(ones_hbm, ones_v)
    pltpu.sync_copy(row2d.at[pl.ds(wid * NCH, NCH)], idx_v)
    plsc.subcore_barrier()

    def body(i, carry):
        pltpu.sync_copy(ones_v, acc_sh.at[idx_v.at[i]], add=True)
        return carry

    lax.fori_loop(0, NCH, body, 0)
    plsc.subcore_barrier()
    pltpu.sync_copy(acc_sh.at[pl.ds(s * RPT, RPT)],
                    out_hbm.at[pl.ds(c * N + s * RPT, RPT)])


_sc_deg = pl.kernel(
    _deg_body,
    out_type=jax.ShapeDtypeStruct((2 * N, 16), jnp.float32),
    mesh=_mesh,
    scratch_types=[
        pltpu.VMEM((NCH, CH), jnp.int32),
        pltpu.VMEM((CH, 16), jnp.float32),
        pltpu.VMEM_SHARED((N, 16), jnp.float32),
    ],
    compiler_params=_sc_params,
)


def _make_gcn(hoff):
    def _gcn_body(xs_hbm, emb_hbm, dis16_hbm, row2d, col2d, zer_hbm, out_hbm,
                  idxr, idxc, xg0, xg1, ev0, ev1, m0, m1, dv0, dv1, acc_sh,
                  sg0, sg1, se0, se1, ss0, ss1, sd0, sd1):
        c = lax.axis_index("c")
        s = lax.axis_index("s")
        wid = c * NS + s

        def zbody(k, carry):
            pltpu.sync_copy(zer_hbm, acc_sh.at[pl.ds(s * RPT + k * CH, CH)])
            return carry

        lax.fori_loop(0, RPT // CH, zbody, 0)
        pltpu.sync_copy(row2d.at[pl.ds(wid * NCH, NCH)], idxr)
        pltpu.sync_copy(col2d.at[pl.ds(wid * NCH, NCH)], idxc)
        plsc.subcore_barrier()

        slots = ((xg0, ev0, m0, dv0, sg0, se0, ss0, sd0),
                 (xg1, ev1, m1, dv1, sg1, se1, ss1, sd1))

        def issue_in(i, slot):
            xg, ev, _, dv, sg, se, _, sd = slot
            pltpu.async_copy(xs_hbm.at[idxr.at[i]], xg, sg)
            pltpu.async_copy(
                emb_hbm.at[pl.ds(wid * EPT + i * CH, CH), pl.ds(hoff, DH)],
                ev, se)
            pltpu.async_copy(dis16_hbm.at[idxr.at[i]], dv, sd)

        def step(i, slot, wait_scatter):
            xg, ev, m, dv, sg, se, ss, sd = slot
            pltpu.make_async_copy(xs_hbm.at[idxr.at[i]], xg, sg).wait()
            pltpu.make_async_copy(
                emb_hbm.at[pl.ds(wid * EPT + i * CH, CH), pl.ds(hoff, DH)],
                ev, se).wait()
            pltpu.make_async_copy(dis16_hbm.at[idxr.at[i]], dv, sd).wait()
            if wait_scatter:
                pltpu.make_async_copy(m, acc_sh.at[idxc.at[i]], ss).wait()

            def rowbody(r, cc):
                d = dv[r, :]
                for j in range(DH // 16):
                    sl = pl.ds(j * 16, 16)
                    m[r, sl] = jnp.maximum((xg[r, sl] + ev[r, sl]) * d, 0.0)
                return cc

            lax.fori_loop(0, CH, rowbody, 0)
            pltpu.async_copy(m, acc_sh.at[idxc.at[i]], ss, add=True)

        issue_in(0, slots[0])
        issue_in(1, slots[1])
        step(0, slots[0], False)
        issue_in(2, slots[0])
        step(1, slots[1], False)
        issue_in(3, slots[1])

        def pair(k, carry):
            i0 = 2 * k
            step(i0, slots[0], True)

            @pl.when(i0 + 2 < NCH)
            def _():
                issue_in(i0 + 2, slots[0])

            step(i0 + 1, slots[1], True)

            @pl.when(i0 + 3 < NCH)
            def _():
                issue_in(i0 + 3, slots[1])

            return carry

        lax.fori_loop(1, NCH // 2, pair, 0)
        pltpu.make_async_copy(m0, acc_sh.at[idxc.at[0]], ss0).wait()
        pltpu.make_async_copy(m1, acc_sh.at[idxc.at[0]], ss1).wait()
        plsc.subcore_barrier()
        pltpu.sync_copy(acc_sh.at[pl.ds(s * RPT, RPT)],
                        out_hbm.at[pl.ds(c * N + s * RPT, RPT)])

    return pl.kernel(
        _gcn_body,
        out_type=jax.ShapeDtypeStruct((2 * N, DH), jnp.float32),
        mesh=_mesh,
        scratch_types=[
            pltpu.VMEM((NCH, CH), jnp.int32),
            pltpu.VMEM((NCH, CH), jnp.int32),
            pltpu.VMEM((CH, DH), jnp.float32),
            pltpu.VMEM((CH, DH), jnp.float32),
            pltpu.VMEM((CH, DH), jnp.float32),
            pltpu.VMEM((CH, DH), jnp.float32),
            pltpu.VMEM((CH, DH), jnp.float32),
            pltpu.VMEM((CH, DH), jnp.float32),
            pltpu.VMEM((CH, 16), jnp.float32),
            pltpu.VMEM((CH, 16), jnp.float32),
            pltpu.VMEM_SHARED((N, DH), jnp.float32),
            pltpu.SemaphoreType.DMA,
            pltpu.SemaphoreType.DMA,
            pltpu.SemaphoreType.DMA,
            pltpu.SemaphoreType.DMA,
            pltpu.SemaphoreType.DMA,
            pltpu.SemaphoreType.DMA,
            pltpu.SemaphoreType.DMA,
            pltpu.SemaphoreType.DMA,
        ],
        compiler_params=_sc_params,
    )


_sc_gcn_lo = _make_gcn(0)
_sc_gcn_hi = _make_gcn(DH)


BNR = 1000
BER = 4000


def _enc_nodemm_body(x_ref, nwT_ref, nb_ref, wT_ref, b_ref,
                     y_ref, ys0_ref, ys1_ref):
    h = jnp.maximum(
        jnp.dot(x_ref[...], nwT_ref[...], preferred_element_type=jnp.float32)
        + nb_ref[...], 0.0)
    y = (jnp.dot(h, wT_ref[...], preferred_element_type=jnp.float32)
         + b_ref[...])
    y_ref[...] = y
    ys0_ref[...] = y[:, :DH]
    ys1_ref[...] = y[:, DH:]


_tc_enc_nodemm = pl.pallas_call(
    _enc_nodemm_body,
    out_shape=(jax.ShapeDtypeStruct((N, D), jnp.float32),
               jax.ShapeDtypeStruct((N, DH), jnp.float32),
               jax.ShapeDtypeStruct((N, DH), jnp.float32)),
    grid=(N // BNR,),
    in_specs=[
        pl.BlockSpec((BNR, D), lambda i: (i, 0)),
        pl.BlockSpec((D, D), lambda i: (0, 0)),
        pl.BlockSpec((1, D), lambda i: (0, 0)),
        pl.BlockSpec((D, D), lambda i: (0, 0)),
        pl.BlockSpec((1, D), lambda i: (0, 0)),
    ],
    out_specs=(pl.BlockSpec((BNR, D), lambda i: (i, 0)),
               pl.BlockSpec((BNR, DH), lambda i: (i, 0)),
               pl.BlockSpec((BNR, DH), lambda i: (i, 0))),
)


def _combine_expr(a0l, a1l, a0h, a1h, y, dis, inv, g, bb):
    acc = jnp.concatenate([a0l + a1l, a0h + a1h], axis=1)
    agg = dis * acc + jnp.maximum(y, 0.0) * inv
    return agg * (g * BN_SCALE) + bb


def _combine_nodemm_body(a0l_ref, a1l_ref, a0h_ref, a1h_ref, y_ref, dis_ref,
                         inv_ref, g_ref, bb_ref, wT_ref, b_ref,
                         yn_ref, ys0_ref, ys1_ref):
    h = jnp.maximum(
        _combine_expr(a0l_ref[...], a1l_ref[...], a0h_ref[...], a1h_ref[...],
                      y_ref[...], dis_ref[...], inv_ref[...], g_ref[...],
                      bb_ref[...]), 0.0)
    y = (jnp.dot(h, wT_ref[...], preferred_element_type=jnp.float32)
         + b_ref[...])
    yn_ref[...] = y
    ys0_ref[...] = y[:, :DH]
    ys1_ref[...] = y[:, DH:]


_tc_combine_nodemm = pl.pallas_call(
    _combine_nodemm_body,
    out_shape=(jax.ShapeDtypeStruct((N, D), jnp.float32),
               jax.ShapeDtypeStruct((N, DH), jnp.float32),
               jax.ShapeDtypeStruct((N, DH), jnp.float32)),
    grid=(N // BNR,),
    in_specs=[
        pl.BlockSpec((BNR, DH), lambda i: (i, 0)),
        pl.BlockSpec((BNR, DH), lambda i: (i, 0)),
        pl.BlockSpec((BNR, DH), lambda i: (i, 0)),
        pl.BlockSpec((BNR, DH), lambda i: (i, 0)),
        pl.BlockSpec((BNR, D), lambda i: (i, 0)),
        pl.BlockSpec((BNR, 1), lambda i: (i, 0)),
        pl.BlockSpec((BNR, 1), lambda i: (i, 0)),
        pl.BlockSpec((1, D), lambda i: (0, 0)),
        pl.BlockSpec((1, D), lambda i: (0, 0)),
        pl.BlockSpec((D, D), lambda i: (0, 0)),
        pl.BlockSpec((1, D), lambda i: (0, 0)),
    ],
    out_specs=(pl.BlockSpec((BNR, D), lambda i: (i, 0)),
               pl.BlockSpec((BNR, DH), lambda i: (i, 0)),
               pl.BlockSpec((BNR, DH), lambda i: (i, 0))),
)


def _edgemm_body(ea_ref, eWT_ref, eb_ref, o_ref):
    o_ref[...] = (jnp.dot(ea_ref[...], eWT_ref[...],
                          preferred_element_type=jnp.float32)
                  + eb_ref[...])


_tc_edgemm = pl.pallas_call(
    _edgemm_body,
    out_shape=jax.ShapeDtypeStruct((E, D), jnp.float32),
    grid=(E // BER,),
    in_specs=[
        pl.BlockSpec((BER, DE), lambda i: (i, 0)),
        pl.BlockSpec((DE, D), lambda i: (0, 0)),
        pl.BlockSpec((1, D), lambda i: (0, 0)),
    ],
    out_specs=pl.BlockSpec((BER, D), lambda i: (i, 0)),
)


def _prep_body(d0_ref, d1_ref, dis_ref, inv_ref, dis16_ref):
    deg = d0_ref[...] + d1_ref[...] + 1.0
    dis16 = lax.rsqrt(deg)
    dis16_ref[...] = dis16
    dis_ref[...] = dis16[:, :1]
    inv_ref[...] = 1.0 / deg[:, :1]


_tc_prep = pl.pallas_call(
    _prep_body,
    out_shape=(jax.ShapeDtypeStruct((N, 1), jnp.float32),
               jax.ShapeDtypeStruct((N, 1), jnp.float32),
               jax.ShapeDtypeStruct((N, 16), jnp.float32)),
    grid=(N // BNR,),
    in_specs=[
        pl.BlockSpec((BNR, 16), lambda i: (i, 0)),
        pl.BlockSpec((BNR, 16), lambda i: (i, 0)),
    ],
    out_specs=(pl.BlockSpec((BNR, 1), lambda i: (i, 0)),
               pl.BlockSpec((BNR, 1), lambda i: (i, 0)),
               pl.BlockSpec((BNR, 16), lambda i: (i, 0))),
)


def _combine_last_body(a0l_ref, a1l_ref, a0h_ref, a1h_ref, y_ref, dis_ref,
                       inv_ref, g_ref, bb_ref, o_ref):
    o_ref[...] = _combine_expr(
        a0l_ref[...], a1l_ref[...], a0h_ref[...], a1h_ref[...], y_ref[...],
        dis_ref[...], inv_ref[...], g_ref[...], bb_ref[...])


_tc_combine_last = pl.pallas_call(
    _combine_last_body,
    out_shape=jax.ShapeDtypeStruct((N, D), jnp.float32),
    grid=(N // BNR,),
    in_specs=[
        pl.BlockSpec((BNR, DH), lambda i: (i, 0)),
        pl.BlockSpec((BNR, DH), lambda i: (i, 0)),
        pl.BlockSpec((BNR, DH), lambda i: (i, 0)),
        pl.BlockSpec((BNR, DH), lambda i: (i, 0)),
        pl.BlockSpec((BNR, D), lambda i: (i, 0)),
        pl.BlockSpec((BNR, 1), lambda i: (i, 0)),
        pl.BlockSpec((BNR, 1), lambda i: (i, 0)),
        pl.BlockSpec((1, D), lambda i: (0, 0)),
        pl.BlockSpec((1, D), lambda i: (0, 0)),
    ],
    out_specs=pl.BlockSpec((BNR, D), lambda i: (i, 0)),
)


def kernel(x, edge_index, edge_attr, node_W, node_b,
           conv0_W, conv0_b, conv0_eW, conv0_eb, bn0_g, bn0_b,
           conv1_W, conv1_b, conv1_eW, conv1_eb, bn1_g, bn1_b,
           conv2_W, conv2_b, conv2_eW, conv2_eb, bn2_g, bn2_b):
    f32 = jnp.float32
    row2d = edge_index[0].astype(jnp.int32).reshape(ROWS2D, CH)
    col2d = edge_index[1].astype(jnp.int32).reshape(ROWS2D, CH)
    ea = edge_attr.astype(f32)
    xf = x.astype(f32)

    ones16 = jnp.ones((CH, 16), f32)
    zer16 = jnp.zeros((CH, 16), f32)
    zerD = jnp.zeros((CH, DH), f32)

    deg2 = _sc_deg(row2d, ones16, zer16)
    dis, inv, dis16 = _tc_prep(deg2[:N], deg2[N:])

    y, ys0, ys1 = _tc_enc_nodemm(xf, node_W.T, node_b.reshape(1, D),
                                 conv0_W.T, conv0_b.reshape(1, D))

    convs = [
        (conv0_eW, conv0_eb, bn0_g, bn0_b, conv1_W, conv1_b),
        (conv1_eW, conv1_eb, bn1_g, bn1_b, conv2_W, conv2_b),
        (conv2_eW, conv2_eb, bn2_g, bn2_b, None, None),
    ]
    for eW, eb, g, bb, Wn, bn in convs:
        emb = _tc_edgemm(ea, eW.T, eb.reshape(1, D))
        acc_lo = _sc_gcn_lo(ys0, emb, dis16, row2d, col2d, zerD)
        acc_hi = _sc_gcn_hi(ys1, emb, dis16, row2d, col2d, zerD)
        if Wn is not None:
            y, ys0, ys1 = _tc_combine_nodemm(
                acc_lo[:N], acc_lo[N:], acc_hi[:N], acc_hi[N:], y, dis, inv,
                g.reshape(1, D), bb.reshape(1, D), Wn.T, bn.reshape(1, D))
        else:
            h = _tc_combine_last(
                acc_lo[:N], acc_lo[N:], acc_hi[:N], acc_hi[N:], y, dis, inv,
                g.reshape(1, D), bb.reshape(1, D))

    return h
